# Initial kernel scaffold; baseline (speedup 1.0000x reference)
import jax
import jax.numpy as jnp
from jax.experimental import pallas as pl


def kernel(continuous, categorical, cont_embedding, cat_tables):
    raise NotImplementedError("write your pallas kernel here")


# speedup vs baseline: 1.3421x; 1.3421x over previous
import functools

import jax
import jax.numpy as jnp
from jax import lax
from jax.experimental import pallas as pl
from jax.experimental.pallas import tpu as pltpu
from jax.experimental.pallas import tpu_sc as plsc

B, S = 1024, 50
CONT_DIM = 13
N_CAT = 26
CARD = 100000
EMB = 16
N_ROWS = CONT_DIM + N_CAT

NC, NS = 2, 16
NW = NC * NS
N = B * S
TOK_PER_W = N // NW
CHUNK = 64
NCHUNK = TOK_PER_W // CHUNK


def _sc_body(idx_hbm, cont_hbm, emb_hbm, table_hbm, out_hbm,
             idx_v, out_v, cont_v, emb_v, sem):
    wid = lax.axis_index("s") * NC + lax.axis_index("c")
    pltpu.sync_copy(emb_hbm, emb_v)

    def chunk_body(k, carry):
        base = wid * TOK_PER_W + k * CHUNK
        pltpu.sync_copy(idx_hbm.at[pl.ds(base, CHUNK), :], idx_v)
        pltpu.sync_copy(cont_hbm.at[pl.ds(base, CHUNK), :], cont_v)

        def tok_body(t, c2):
            pltpu.async_copy(
                table_hbm.at[idx_v.at[t]],
                out_v.at[t, pl.ds(CONT_DIM, N_CAT), :],
                sem,
            )
            idx_t = jnp.full((EMB,), t, dtype=jnp.int32)
            for c in range(CONT_DIM):
                s_vec = plsc.load_gather(
                    cont_v, [idx_t, jnp.full((EMB,), c, dtype=jnp.int32)])
                out_v[t, c, :] = emb_v[c, :] * s_vec
            return c2

        lax.fori_loop(0, CHUNK, tok_body, 0)

        def drain_body(t, c2):
            pltpu.make_async_copy(
                table_hbm.at[idx_v.at[t]],
                out_v.at[t, pl.ds(CONT_DIM, N_CAT), :],
                sem,
            ).wait()
            return c2

        lax.fori_loop(0, CHUNK, drain_body, 0)
        pltpu.sync_copy(out_v, out_hbm.at[pl.ds(base, CHUNK), :, :])
        return carry

    lax.fori_loop(0, NCHUNK, chunk_body, 0)


@jax.jit
def _sc_embed(flat_idx, cont_flat, cont_embedding, table):
    mesh = plsc.VectorSubcoreMesh(
        core_axis_name="c", subcore_axis_name="s",
        num_cores=NC, num_subcores=NS,
    )
    run = pl.kernel(
        _sc_body,
        out_type=jax.ShapeDtypeStruct((N, N_ROWS, EMB), jnp.float32),
        mesh=mesh,
        scratch_types=[
            pltpu.VMEM((CHUNK, N_CAT), jnp.int32),
            pltpu.VMEM((CHUNK, N_ROWS, EMB), jnp.float32),
            pltpu.VMEM((CHUNK, CONT_DIM), jnp.float32),
            pltpu.VMEM((CONT_DIM, EMB), jnp.float32),
            pltpu.SemaphoreType.DMA,
        ],
        compiler_params=pltpu.CompilerParams(
            use_tc_tiling_on_sc=False, needs_layout_passes=False),
    )
    return run(flat_idx, cont_flat, cont_embedding, table)


def kernel(continuous, categorical, cont_embedding, cat_tables):
    cat_flat = categorical.reshape(N, N_CAT)
    flat_idx = cat_flat + (jnp.arange(N_CAT, dtype=jnp.int32) * CARD)[None, :]
    cont_flat = continuous.reshape(N, CONT_DIM)
    table = cat_tables.reshape(N_CAT * CARD, EMB)
    out = _sc_embed(flat_idx, cont_flat, cont_embedding, table)
    return out.reshape(B, S, N_ROWS, EMB)


# speedup vs baseline: 5.8892x; 4.3880x over previous
import functools

import jax
import jax.numpy as jnp
from jax import lax
from jax.experimental import pallas as pl
from jax.experimental.pallas import tpu as pltpu
from jax.experimental.pallas import tpu_sc as plsc

B, S = 1024, 50
CONT_DIM = 13
N_CAT = 26
CARD = 100000
EMB = 16
N_ROWS = CONT_DIM + N_CAT

NC, NS = 2, 16
NW = NC * NS
GU_PER_W = (N_CAT * EMB) // NW
CU = CONT_DIM * EMB
CU_ROUNDS = (CU + NW - 1) // NW
LANES = 16
VECS = B // LANES


def _sc_body(cat_t, cont_t, emb_f, tab_t, out4,
             row_v, idx_v, val_v, cval_v, emb_v, sem_i, sem_o):
    wid = lax.axis_index("s") * NC + lax.axis_index("c")
    pltpu.sync_copy(emb_f, emb_v)

    def g_unit(j, carry):
        g = wid * GU_PER_W + j
        f = g // EMB
        e = g % EMB
        pltpu.sync_copy(tab_t.at[f, e, :], row_v)
        pltpu.async_copy(cat_t.at[f, 0, :], idx_v.at[0], sem_i)
        pltpu.async_copy(cat_t.at[f, 1, :], idx_v.at[1], sem_i)

        def s_body(s, c2):
            sl = lax.rem(s, 2)
            pltpu.make_async_copy(cat_t.at[f, s, :], idx_v.at[sl], sem_i).wait()

            @pl.when(s >= 2)
            def _():
                pltpu.make_async_copy(val_v.at[sl], out4.at[s, 0, 0, :],
                                      sem_o).wait()

            for i in range(VECS):
                iv = idx_v[sl, pl.ds(i * LANES, LANES)]
                val_v[sl, pl.ds(i * LANES, LANES)] = plsc.load_gather(
                    row_v, [iv])

            pltpu.async_copy(val_v.at[sl], out4.at[s, CONT_DIM + f, e, :],
                             sem_o)

            @pl.when(s + 2 < S)
            def _():
                pltpu.async_copy(cat_t.at[f, s + 2, :], idx_v.at[sl], sem_i)

            return c2

        lax.fori_loop(0, S, s_body, 0)
        pltpu.make_async_copy(val_v.at[0], out4.at[0, 0, 0, :], sem_o).wait()
        pltpu.make_async_copy(val_v.at[1], out4.at[0, 0, 0, :], sem_o).wait()
        return carry

    lax.fori_loop(0, GU_PER_W, g_unit, 0)

    def c_unit(j, carry):
        u = wid + NW * j

        @pl.when(u < CU)
        def _():
            c = u // EMB
            e = u % EMB
            scal = plsc.load_gather(
                emb_v, [jnp.full((LANES,), c * EMB + e, dtype=jnp.int32)])
            pltpu.async_copy(cont_t.at[c, 0, :], cval_v.at[0], sem_i)
            pltpu.async_copy(cont_t.at[c, 1, :], cval_v.at[1], sem_i)

            def s_body(s, c2):
                sl = lax.rem(s, 2)
                pltpu.make_async_copy(cont_t.at[c, s, :], cval_v.at[sl],
                                      sem_i).wait()

                @pl.when(s >= 2)
                def _():
                    pltpu.make_async_copy(val_v.at[sl], out4.at[s, 0, 0, :],
                                          sem_o).wait()

                for i in range(VECS):
                    sli = pl.ds(i * LANES, LANES)
                    val_v[sl, sli] = cval_v[sl, sli] * scal

                pltpu.async_copy(val_v.at[sl], out4.at[s, c, e, :], sem_o)

                @pl.when(s + 2 < S)
                def _():
                    pltpu.async_copy(cont_t.at[c, s + 2, :], cval_v.at[sl],
                                     sem_i)

                return c2

            lax.fori_loop(0, S, s_body, 0)
            pltpu.make_async_copy(val_v.at[0], out4.at[0, 0, 0, :],
                                  sem_o).wait()
            pltpu.make_async_copy(val_v.at[1], out4.at[0, 0, 0, :],
                                  sem_o).wait()

        return carry

    lax.fori_loop(0, CU_ROUNDS, c_unit, 0)


@jax.jit
def _sc_embed(cat_t, cont_t, emb_f, tab_t):
    mesh = plsc.VectorSubcoreMesh(
        core_axis_name="c", subcore_axis_name="s",
        num_cores=NC, num_subcores=NS,
    )
    run = pl.kernel(
        _sc_body,
        out_type=jax.ShapeDtypeStruct((S, N_ROWS, EMB, B), jnp.float32),
        mesh=mesh,
        scratch_types=[
            pltpu.VMEM((CARD,), jnp.float32),
            pltpu.VMEM((2, B), jnp.int32),
            pltpu.VMEM((2, B), jnp.float32),
            pltpu.VMEM((2, B), jnp.float32),
            pltpu.VMEM((CONT_DIM * EMB,), jnp.float32),
            pltpu.SemaphoreType.DMA,
            pltpu.SemaphoreType.DMA,
        ],
        compiler_params=pltpu.CompilerParams(
            use_tc_tiling_on_sc=True, needs_layout_passes=False),
    )
    return run(cat_t, cont_t, emb_f, tab_t)


def kernel(continuous, categorical, cont_embedding, cat_tables):
    cat_t = jnp.transpose(categorical, (2, 1, 0))
    cont_t = jnp.transpose(continuous, (2, 1, 0))
    tab_t = jnp.transpose(cat_tables, (0, 2, 1))
    emb_f = cont_embedding.reshape(CONT_DIM * EMB)
    out4 = _sc_embed(cat_t, cont_t, emb_f, tab_t)
    return jnp.transpose(out4, (3, 0, 1, 2))


# speedup vs baseline: 8.8195x; 1.4976x over previous
import functools

import jax
import jax.numpy as jnp
from jax import lax
from jax.experimental import pallas as pl
from jax.experimental.pallas import tpu as pltpu
from jax.experimental.pallas import tpu_sc as plsc

B, S = 1024, 50
CONT_DIM = 13
N_CAT = 26
CARD = 100000
EMB = 16
N_ROWS = CONT_DIM + N_CAT

NC, NS = 2, 16
NW = NC * NS
GU_PER_W = (N_CAT * EMB) // NW
CU = CONT_DIM * EMB
CU_ROUNDS = (CU + NW - 1) // NW
LANES = 16
D = 8


def _sc_body(cat_t, cont_t, emb_f, tab_t, out4,
             row_v, idx_v, val_v, cval_v, emb_v, sem_r, sem_i, sem_o):
    wid = lax.axis_index("s") * NC + lax.axis_index("c")
    pltpu.sync_copy(emb_f, emb_v)

    def row_copy(j, sem):
        g = wid * GU_PER_W + j
        return pltpu.make_async_copy(
            tab_t.at[g // EMB, g % EMB, :], row_v, sem)

    row_copy(0, sem_r).start()

    def g_unit(j, carry):
        g = wid * GU_PER_W + j
        f = g // EMB
        e = g % EMB
        row_copy(j, sem_r).wait()
        for q in range(D):
            pltpu.async_copy(cat_t.at[f, q, :], idx_v.at[q], sem_i)

        def s_body(s, c2):
            sl = lax.rem(s, D)
            pltpu.make_async_copy(cat_t.at[f, s, :], idx_v.at[sl],
                                  sem_i).wait()

            @pl.when(s >= D)
            def _():
                pltpu.make_async_copy(val_v.at[sl], out4.at[0, 0, 0, :],
                                      sem_o).wait()

            for i in range(B // LANES):
                sli = pl.ds(i * LANES, LANES)
                val_v[sl, sli] = plsc.load_gather(row_v, [idx_v[sl, sli]])

            pltpu.async_copy(val_v.at[sl], out4.at[s, CONT_DIM + f, e, :],
                             sem_o)

            @pl.when(s + D < S)
            def _():
                pltpu.async_copy(cat_t.at[f, s + D, :], idx_v.at[sl], sem_i)

            return c2

        lax.fori_loop(0, S, s_body, 0)

        @pl.when(j + 1 < GU_PER_W)
        def _():
            row_copy(j + 1, sem_r).start()

        def g_drain(q, c2):
            pltpu.make_async_copy(val_v.at[q], out4.at[0, 0, 0, :],
                                  sem_o).wait()
            return c2

        lax.fori_loop(0, D, g_drain, 0)

        @pl.when(j < CU_ROUNDS)
        def _():
            u = wid + NW * j

            @pl.when(u < CU)
            def _():
                c = u // EMB
                e2 = u % EMB
                scal = plsc.load_gather(
                    emb_v, [jnp.full((LANES,), c * EMB + e2,
                                     dtype=jnp.int32)])
                for q in range(D):
                    pltpu.async_copy(cont_t.at[c, q, :], cval_v.at[q],
                                     sem_i)

                def cs_body(s, c2):
                    sl = lax.rem(s, D)
                    pltpu.make_async_copy(cont_t.at[c, s, :],
                                          cval_v.at[sl], sem_i).wait()

                    @pl.when(s >= D)
                    def _():
                        pltpu.make_async_copy(val_v.at[sl],
                                              out4.at[0, 0, 0, :],
                                              sem_o).wait()

                    for i in range(B // LANES):
                        sli = pl.ds(i * LANES, LANES)
                        val_v[sl, sli] = cval_v[sl, sli] * scal

                    pltpu.async_copy(val_v.at[sl], out4.at[s, c, e2, :],
                                     sem_o)

                    @pl.when(s + D < S)
                    def _():
                        pltpu.async_copy(cont_t.at[c, s + D, :],
                                         cval_v.at[sl], sem_i)

                    return c2

                lax.fori_loop(0, S, cs_body, 0)

                def c_drain(q, c2):
                    pltpu.make_async_copy(val_v.at[q],
                                          out4.at[0, 0, 0, :],
                                          sem_o).wait()
                    return c2

                lax.fori_loop(0, D, c_drain, 0)

        return carry

    lax.fori_loop(0, GU_PER_W, g_unit, 0)


@jax.jit
def _sc_embed(cat_t, cont_t, emb_f, tab_t):
    mesh = plsc.VectorSubcoreMesh(
        core_axis_name="c", subcore_axis_name="s",
        num_cores=NC, num_subcores=NS,
    )
    run = pl.kernel(
        _sc_body,
        out_type=jax.ShapeDtypeStruct((S, N_ROWS, EMB, B), jnp.float32),
        mesh=mesh,
        scratch_types=[
            pltpu.VMEM((CARD,), jnp.float32),
            pltpu.VMEM((D, B), jnp.int32),
            pltpu.VMEM((D, B), jnp.float32),
            pltpu.VMEM((D, B), jnp.float32),
            pltpu.VMEM((CONT_DIM * EMB,), jnp.float32),
            pltpu.SemaphoreType.DMA,
            pltpu.SemaphoreType.DMA,
            pltpu.SemaphoreType.DMA,
        ],
        compiler_params=pltpu.CompilerParams(
            use_tc_tiling_on_sc=True, needs_layout_passes=False),
    )
    return run(cat_t, cont_t, emb_f, tab_t)


def kernel(continuous, categorical, cont_embedding, cat_tables):
    cat_t = jnp.transpose(categorical, (2, 1, 0))
    cont_t = jnp.transpose(continuous, (2, 1, 0))
    tab_t = jnp.transpose(cat_tables, (0, 2, 1))
    emb_f = cont_embedding.reshape(CONT_DIM * EMB)
    out4 = _sc_embed(cat_t, cont_t, emb_f, tab_t)
    return jnp.transpose(out4, (3, 0, 1, 2))
